# baseline (speedup 1.0000x reference)
import jax
import jax.numpy as jnp
from jax.experimental import pallas as pl
from jax.experimental.pallas import tpu as pltpu

HIDDEN = 128
N_AA = 20
PEP_LENGTH = 15
MAX_TCR_LEN = 27
VOCAB = N_AA + 1
ENC_DIM = 32 + N_AA
VOC_PAD = 32
ENC_PAD = 64
G4 = 4 * HIDDEN
NB = 4


def _sig(x):
    return 0.5 + 0.5 * jnp.tanh(0.5 * x)


def _cell(gates, c):
    i = _sig(gates[:HIDDEN])
    f = _sig(gates[HIDDEN:2 * HIDDEN])
    g = jnp.tanh(gates[2 * HIDDEN:3 * HIDDEN])
    o = _sig(gates[3 * HIDDEN:])
    c_new = f * c + i * g
    h_new = o * jnp.tanh(c_new)
    return h_new, c_new


def _dot(a, b):
    return jnp.dot(a, b, preferred_element_type=jnp.float32)


def _seq_kernel(pep_t3_ref, tcr_t3_ref, encT_ref,
                wih_pf_ref, wih_pb_ref, wih_tf_ref, wih_tb_ref,
                whh_pf_ref, whh_pb_ref, whh_tf_ref, whh_tb_ref,
                h0p_ref, c0p_ref, h0t_ref, c0t_ref,
                tcr_out_ref, tcr_hn_ref, pep_emb_ref,
                oh_pep_ref, oh_tcr_ref, tcr_scr_ref):
    encT = encT_ref[...]
    tbl_pf = _dot(wih_pf_ref[...], encT)
    tbl_pb = _dot(wih_pb_ref[...], encT)
    tbl_tf = _dot(wih_tf_ref[...], encT)
    tbl_tb = _dot(wih_tb_ref[...], encT)

    Bb = pep_t3_ref.shape[2]
    iota_p = jax.lax.broadcasted_iota(jnp.int32, (PEP_LENGTH, VOC_PAD, Bb), 1)
    oh_pep_ref[...] = (pep_t3_ref[...] == iota_p).astype(jnp.float32)
    iota_t = jax.lax.broadcasted_iota(jnp.int32, (MAX_TCR_LEN, VOC_PAD, Bb), 1)
    oh_tcr_ref[...] = (tcr_t3_ref[...] == iota_t).astype(jnp.float32)

    lens_p = jnp.sum((pep_t3_ref[:, 0, :] != 0).astype(jnp.int32), axis=0,
                     keepdims=True)
    lens_t = jnp.sum((tcr_t3_ref[:, 0, :] != 0).astype(jnp.int32), axis=0,
                     keepdims=True)

    def cell_step(oh, tbl, w, h, c, m):
        g = _dot(tbl, oh) + _dot(w, h)
        h_new, c_new = _cell(g, c)
        return jnp.where(m, h_new, h), jnp.where(m, c_new, c), h_new

    wpf, wpb = whh_pf_ref[...], whh_pb_ref[...]
    wtf, wtb = whh_tf_ref[...], whh_tb_ref[...]

    def tcr_step(i, hft, cft, hbt, cbt):
        tb = MAX_TCR_LEN - 1 - i
        mf = i < lens_t
        mb = tb < lens_t
        hft, cft, hf_new = cell_step(oh_tcr_ref[i], tbl_tf, wtf, hft, cft, mf)
        hbt, cbt, hb_new = cell_step(oh_tcr_ref[tb], tbl_tb, wtb, hbt, cbt, mb)
        tcr_scr_ref[i, :HIDDEN, :] = jnp.where(mf, hf_new, 0.0)
        tcr_scr_ref[tb, HIDDEN:, :] = jnp.where(mb, hb_new, 0.0)
        return hft, cft, hbt, cbt

    def body_a(i, carry):
        hfp, cfp, hbp, cbp, hft, cft, hbt, cbt = carry
        tb = PEP_LENGTH - 1 - i
        hfp, cfp, _ = cell_step(oh_pep_ref[i], tbl_pf, wpf, hfp, cfp,
                                i < lens_p)
        hbp, cbp, _ = cell_step(oh_pep_ref[tb], tbl_pb, wpb, hbp, cbp,
                                tb < lens_p)
        hft, cft, hbt, cbt = tcr_step(i, hft, cft, hbt, cbt)
        return hfp, cfp, hbp, cbp, hft, cft, hbt, cbt

    def body_b(i, carry):
        hft, cft, hbt, cbt = carry
        return tcr_step(i, hft, cft, hbt, cbt)

    h0p, c0p = h0p_ref[...].T, c0p_ref[...].T
    h0t, c0t = h0t_ref[...].T, c0t_ref[...].T
    hfp, _, hbp, _, hft, cft, hbt, cbt = jax.lax.fori_loop(
        0, PEP_LENGTH, body_a,
        (h0p, c0p, h0p, c0p, h0t, c0t, h0t, c0t), unroll=3)
    hft, _, hbt, _ = jax.lax.fori_loop(
        PEP_LENGTH, MAX_TCR_LEN, body_b, (hft, cft, hbt, cbt), unroll=3)

    pep_emb_ref[...] = jnp.concatenate([hfp, hbp], axis=0).T
    tcr_hn_ref[0] = hft.T
    tcr_hn_ref[1] = hbt.T
    tcr_out_ref[...] = jnp.transpose(tcr_scr_ref[...], (2, 0, 1))


def _prep_w(wih, b):
    w = jnp.zeros((G4, ENC_PAD), jnp.float32)
    return w.at[:, :ENC_DIM].set(wih).at[:, ENC_DIM].set(b)


@jax.jit
def kernel(obs, emb_table, onehot_dict, pep_Wih_f, pep_Whh_f, pep_b_f,
           pep_Wih_b, pep_Whh_b, pep_b_b, tcr_Wih_f, tcr_Whh_f, tcr_b_f,
           tcr_Wih_b, tcr_Whh_b, tcr_b_b, h0_pep, c0_pep, h0_tcr, c0_tcr):
    B = obs.shape[0]
    Bb = B // NB
    obs = obs.astype(jnp.int32)
    tcr_t3 = obs[:, :MAX_TCR_LEN].T.reshape(MAX_TCR_LEN, 1, B)
    pep_t3 = obs[:, MAX_TCR_LEN:].T.reshape(PEP_LENGTH, 1, B)

    encT = jnp.zeros((ENC_PAD, VOC_PAD), jnp.float32)
    encT = encT.at[:ENC_DIM, :VOCAB].set(
        jnp.concatenate([emb_table, onehot_dict], axis=1).T)
    encT = encT.at[ENC_DIM, :].set(1.0)

    args = (pep_t3, tcr_t3, encT,
            _prep_w(pep_Wih_f, pep_b_f), _prep_w(pep_Wih_b, pep_b_b),
            _prep_w(tcr_Wih_f, tcr_b_f), _prep_w(tcr_Wih_b, tcr_b_b),
            pep_Whh_f, pep_Whh_b, tcr_Whh_f, tcr_Whh_b,
            h0_pep, c0_pep, h0_tcr, c0_tcr)

    full = lambda b: (0, 0)
    bat2 = lambda b: (0, b)
    bat3 = lambda b: (0, 0, b)
    in_specs = [
        pl.BlockSpec((PEP_LENGTH, 1, Bb), bat3),
        pl.BlockSpec((MAX_TCR_LEN, 1, Bb), bat3),
        pl.BlockSpec((ENC_PAD, VOC_PAD), full),
        pl.BlockSpec((G4, ENC_PAD), full),
        pl.BlockSpec((G4, ENC_PAD), full),
        pl.BlockSpec((G4, ENC_PAD), full),
        pl.BlockSpec((G4, ENC_PAD), full),
        pl.BlockSpec((G4, HIDDEN), full),
        pl.BlockSpec((G4, HIDDEN), full),
        pl.BlockSpec((G4, HIDDEN), full),
        pl.BlockSpec((G4, HIDDEN), full),
        pl.BlockSpec((Bb, HIDDEN), lambda b: (b, 0)),
        pl.BlockSpec((Bb, HIDDEN), lambda b: (b, 0)),
        pl.BlockSpec((Bb, HIDDEN), lambda b: (b, 0)),
        pl.BlockSpec((Bb, HIDDEN), lambda b: (b, 0)),
    ]
    out_specs = [
        pl.BlockSpec((Bb, MAX_TCR_LEN, 2 * HIDDEN), lambda b: (b, 0, 0)),
        pl.BlockSpec((2, Bb, HIDDEN), lambda b: (0, b, 0)),
        pl.BlockSpec((Bb, 2 * HIDDEN), lambda b: (b, 0)),
    ]
    out_shapes = [
        jax.ShapeDtypeStruct((B, MAX_TCR_LEN, 2 * HIDDEN), jnp.float32),
        jax.ShapeDtypeStruct((2, B, HIDDEN), jnp.float32),
        jax.ShapeDtypeStruct((B, 2 * HIDDEN), jnp.float32),
    ]
    tcr_out, tcr_hn, pep_emb = pl.pallas_call(
        _seq_kernel,
        grid=(NB,),
        in_specs=in_specs,
        out_specs=out_specs,
        out_shape=out_shapes,
        scratch_shapes=[
            pltpu.VMEM((PEP_LENGTH, VOC_PAD, Bb), jnp.float32),
            pltpu.VMEM((MAX_TCR_LEN, VOC_PAD, Bb), jnp.float32),
            pltpu.VMEM((MAX_TCR_LEN, 2 * HIDDEN, Bb), jnp.float32),
        ],
        compiler_params=pltpu.CompilerParams(
            dimension_semantics=("parallel",)),
    )(*args)
    return tcr_out, tcr_hn, pep_emb


# speedup vs baseline: 1.0103x; 1.0103x over previous
import jax
import jax.numpy as jnp
from jax.experimental import pallas as pl
from jax.experimental.pallas import tpu as pltpu

HIDDEN = 128
N_AA = 20
PEP_LENGTH = 15
MAX_TCR_LEN = 27
VOCAB = N_AA + 1
ENC_DIM = 32 + N_AA
VOC_PAD = 32
ENC_PAD = 64
G4 = 4 * HIDDEN
NB = 2


def _sig(x):
    return 0.5 + 0.5 * jnp.tanh(0.5 * x)


def _cell(gates, c):
    i = _sig(gates[:, :HIDDEN])
    f = _sig(gates[:, HIDDEN:2 * HIDDEN])
    g = jnp.tanh(gates[:, 2 * HIDDEN:3 * HIDDEN])
    o = _sig(gates[:, 3 * HIDDEN:])
    c_new = f * c + i * g
    h_new = o * jnp.tanh(c_new)
    return h_new, c_new


def _dot(a, b):
    return jnp.dot(a, b, preferred_element_type=jnp.float32)


def _seq_kernel(pep_tok_ref, tcr_tok_ref, enc_ref,
                wih_pf_ref, wih_pb_ref, wih_tf_ref, wih_tb_ref,
                whh_pf_ref, whh_pb_ref, whh_tf_ref, whh_tb_ref,
                h0p_ref, c0p_ref, h0t_ref, c0t_ref,
                tcr_out_ref, tcr_hn_ref, pep_emb_ref,
                oh_pep_ref, oh_tcr_ref):
    enc = enc_ref[...]
    tbl_pf = _dot(enc, wih_pf_ref[...])
    tbl_pb = _dot(enc, wih_pb_ref[...])
    tbl_tf = _dot(enc, wih_tf_ref[...])
    tbl_tb = _dot(enc, wih_tb_ref[...])

    Bb = pep_tok_ref.shape[0]
    pep3 = pep_tok_ref[...].T.reshape(PEP_LENGTH, Bb, 1)
    iota_p = jax.lax.broadcasted_iota(jnp.int32, (PEP_LENGTH, Bb, VOC_PAD), 2)
    oh_pep_ref[...] = (pep3 == iota_p).astype(jnp.float32)
    tcr3 = tcr_tok_ref[...].T.reshape(MAX_TCR_LEN, Bb, 1)
    iota_t = jax.lax.broadcasted_iota(jnp.int32, (MAX_TCR_LEN, Bb, VOC_PAD), 2)
    oh_tcr_ref[...] = (tcr3 == iota_t).astype(jnp.float32)

    lens_p = jnp.sum((pep_tok_ref[...] != 0).astype(jnp.int32), axis=1,
                     keepdims=True)
    lens_t = jnp.sum((tcr_tok_ref[...] != 0).astype(jnp.int32), axis=1,
                     keepdims=True)

    def cell_step(oh, tbl, w, h, c, m):
        g = _dot(oh, tbl) + _dot(h, w)
        h_new, c_new = _cell(g, c)
        return jnp.where(m, h_new, h), jnp.where(m, c_new, c), h_new

    wpf, wpb = whh_pf_ref[...], whh_pb_ref[...]
    wtf, wtb = whh_tf_ref[...], whh_tb_ref[...]

    def tcr_step(i, hft, cft, hbt, cbt):
        tb = MAX_TCR_LEN - 1 - i
        mf = i < lens_t
        mb = tb < lens_t
        hft, cft, hf_new = cell_step(oh_tcr_ref[i], tbl_tf, wtf, hft, cft, mf)
        hbt, cbt, hb_new = cell_step(oh_tcr_ref[tb], tbl_tb, wtb, hbt, cbt, mb)
        tcr_out_ref[:, i, :HIDDEN] = jnp.where(mf, hf_new, 0.0)
        tcr_out_ref[:, tb, HIDDEN:] = jnp.where(mb, hb_new, 0.0)
        return hft, cft, hbt, cbt

    def body_a(i, carry):
        hfp, cfp, hbp, cbp, hft, cft, hbt, cbt = carry
        tb = PEP_LENGTH - 1 - i
        hfp, cfp, _ = cell_step(oh_pep_ref[i], tbl_pf, wpf, hfp, cfp,
                                i < lens_p)
        hbp, cbp, _ = cell_step(oh_pep_ref[tb], tbl_pb, wpb, hbp, cbp,
                                tb < lens_p)
        hft, cft, hbt, cbt = tcr_step(i, hft, cft, hbt, cbt)
        return hfp, cfp, hbp, cbp, hft, cft, hbt, cbt

    def body_b(i, carry):
        hft, cft, hbt, cbt = carry
        return tcr_step(i, hft, cft, hbt, cbt)

    h0p, c0p = h0p_ref[...], c0p_ref[...]
    h0t, c0t = h0t_ref[...], c0t_ref[...]
    carry = (h0p, c0p, h0p, c0p, h0t, c0t, h0t, c0t)
    for i in range(PEP_LENGTH):
        carry = body_a(i, carry)
    hfp, _, hbp, _, hft, cft, hbt, cbt = carry
    carry = (hft, cft, hbt, cbt)
    for i in range(PEP_LENGTH, MAX_TCR_LEN):
        carry = body_b(i, carry)
    hft, _, hbt, _ = carry

    pep_emb_ref[:, :HIDDEN] = hfp
    pep_emb_ref[:, HIDDEN:] = hbp
    tcr_hn_ref[0] = hft
    tcr_hn_ref[1] = hbt


def _prep_w(wih, b):
    w = jnp.zeros((ENC_PAD, G4), jnp.float32)
    return w.at[:ENC_DIM].set(wih.T).at[ENC_DIM].set(b)


@jax.jit
def kernel(obs, emb_table, onehot_dict, pep_Wih_f, pep_Whh_f, pep_b_f,
           pep_Wih_b, pep_Whh_b, pep_b_b, tcr_Wih_f, tcr_Whh_f, tcr_b_f,
           tcr_Wih_b, tcr_Whh_b, tcr_b_b, h0_pep, c0_pep, h0_tcr, c0_tcr):
    B = obs.shape[0]
    Bb = B // NB
    obs = obs.astype(jnp.int32)
    tcr_tok = obs[:, :MAX_TCR_LEN]
    pep_tok = obs[:, MAX_TCR_LEN:]

    enc = jnp.zeros((VOC_PAD, ENC_PAD), jnp.float32)
    enc = enc.at[:VOCAB, :ENC_DIM].set(
        jnp.concatenate([emb_table, onehot_dict], axis=1))
    enc = enc.at[:VOCAB, ENC_DIM].set(1.0)

    args = (pep_tok, tcr_tok, enc,
            _prep_w(pep_Wih_f, pep_b_f), _prep_w(pep_Wih_b, pep_b_b),
            _prep_w(tcr_Wih_f, tcr_b_f), _prep_w(tcr_Wih_b, tcr_b_b),
            pep_Whh_f.T, pep_Whh_b.T, tcr_Whh_f.T, tcr_Whh_b.T,
            h0_pep, c0_pep, h0_tcr, c0_tcr)

    full = lambda b: (0, 0)
    bat2 = lambda b: (b, 0)
    in_specs = [
        pl.BlockSpec((Bb, PEP_LENGTH), bat2),
        pl.BlockSpec((Bb, MAX_TCR_LEN), bat2),
        pl.BlockSpec((VOC_PAD, ENC_PAD), full),
        pl.BlockSpec((ENC_PAD, G4), full),
        pl.BlockSpec((ENC_PAD, G4), full),
        pl.BlockSpec((ENC_PAD, G4), full),
        pl.BlockSpec((ENC_PAD, G4), full),
        pl.BlockSpec((HIDDEN, G4), full),
        pl.BlockSpec((HIDDEN, G4), full),
        pl.BlockSpec((HIDDEN, G4), full),
        pl.BlockSpec((HIDDEN, G4), full),
        pl.BlockSpec((Bb, HIDDEN), bat2),
        pl.BlockSpec((Bb, HIDDEN), bat2),
        pl.BlockSpec((Bb, HIDDEN), bat2),
        pl.BlockSpec((Bb, HIDDEN), bat2),
    ]
    out_specs = [
        pl.BlockSpec((Bb, MAX_TCR_LEN, 2 * HIDDEN), lambda b: (b, 0, 0)),
        pl.BlockSpec((2, Bb, HIDDEN), lambda b: (0, b, 0)),
        pl.BlockSpec((Bb, 2 * HIDDEN), bat2),
    ]
    out_shapes = [
        jax.ShapeDtypeStruct((B, MAX_TCR_LEN, 2 * HIDDEN), jnp.float32),
        jax.ShapeDtypeStruct((2, B, HIDDEN), jnp.float32),
        jax.ShapeDtypeStruct((B, 2 * HIDDEN), jnp.float32),
    ]
    tcr_out, tcr_hn, pep_emb = pl.pallas_call(
        _seq_kernel,
        grid=(NB,),
        in_specs=in_specs,
        out_specs=out_specs,
        out_shape=out_shapes,
        scratch_shapes=[
            pltpu.VMEM((PEP_LENGTH, Bb, VOC_PAD), jnp.float32),
            pltpu.VMEM((MAX_TCR_LEN, Bb, VOC_PAD), jnp.float32),
        ],
        compiler_params=pltpu.CompilerParams(
            dimension_semantics=("arbitrary",)),
    )(*args)
    return tcr_out, tcr_hn, pep_emb


# speedup vs baseline: 1.1627x; 1.1508x over previous
import jax
import jax.numpy as jnp
from jax.experimental import pallas as pl
from jax.experimental.pallas import tpu as pltpu

HIDDEN = 128
N_AA = 20
PEP_LENGTH = 15
MAX_TCR_LEN = 27
TOT_LEN = MAX_TCR_LEN + PEP_LENGTH
VOCAB = N_AA + 1
ENC_DIM = 32 + N_AA
VOC_PAD = 32
G4 = 4 * HIDDEN
NB = 2


def _sig(x):
    return 0.5 + 0.5 * jnp.tanh(0.5 * x)


def _cell(gates, c):
    i = _sig(gates[:, :HIDDEN])
    f = _sig(gates[:, HIDDEN:2 * HIDDEN])
    g = jnp.tanh(gates[:, 2 * HIDDEN:3 * HIDDEN])
    o = _sig(gates[:, 3 * HIDDEN:])
    c_new = f * c + i * g
    h_new = o * jnp.tanh(c_new)
    return h_new, c_new


def _dot(a, b):
    return jnp.dot(a, b, preferred_element_type=jnp.float32)


def _seq_kernel(obs_ref, emb_ref, onehot_ref,
                wih_pf_ref, wih_pb_ref, wih_tf_ref, wih_tb_ref,
                b_pf_ref, b_pb_ref, b_tf_ref, b_tb_ref,
                whh_pf_ref, whh_pb_ref, whh_tf_ref, whh_tb_ref,
                h0p_ref, c0p_ref, h0t_ref, c0t_ref,
                tcr_out_ref, tcr_hn_ref, pep_emb_ref,
                oh_pep_ref, oh_tcr_ref):
    enc = jnp.concatenate([emb_ref[...], onehot_ref[...]], axis=1)
    enc = jnp.pad(enc, ((0, VOC_PAD - VOCAB), (0, 0)))

    def tbl(wih_ref, b_ref):
        return _dot(enc, wih_ref[...].T) + b_ref[...]

    tbl_pf = tbl(wih_pf_ref, b_pf_ref)
    tbl_pb = tbl(wih_pb_ref, b_pb_ref)
    tbl_tf = tbl(wih_tf_ref, b_tf_ref)
    tbl_tb = tbl(wih_tb_ref, b_tb_ref)

    tcr_tok = obs_ref[:, :MAX_TCR_LEN]
    pep_tok = obs_ref[:, MAX_TCR_LEN:]

    Bb = obs_ref.shape[0]
    pep3 = pep_tok.T.reshape(PEP_LENGTH, Bb, 1)
    iota_p = jax.lax.broadcasted_iota(jnp.int32, (PEP_LENGTH, Bb, VOC_PAD), 2)
    oh_pep_ref[...] = (pep3 == iota_p).astype(jnp.float32)
    tcr3 = tcr_tok.T.reshape(MAX_TCR_LEN, Bb, 1)
    iota_t = jax.lax.broadcasted_iota(jnp.int32, (MAX_TCR_LEN, Bb, VOC_PAD), 2)
    oh_tcr_ref[...] = (tcr3 == iota_t).astype(jnp.float32)

    lens_p = jnp.sum((pep_tok != 0).astype(jnp.int32), axis=1,
                     keepdims=True)
    lens_t = jnp.sum((tcr_tok != 0).astype(jnp.int32), axis=1, keepdims=True)

    def cell_step(oh, tbl_d, w, h, c, m):
        g = _dot(oh, tbl_d) + _dot(h, w)
        h_new, c_new = _cell(g, c)
        return jnp.where(m, h_new, h), jnp.where(m, c_new, c), h_new

    wpf, wpb = whh_pf_ref[...].T, whh_pb_ref[...].T
    wtf, wtb = whh_tf_ref[...].T, whh_tb_ref[...].T

    def tcr_step(i, hft, cft, hbt, cbt):
        tb = MAX_TCR_LEN - 1 - i
        mf = i < lens_t
        mb = tb < lens_t
        hft, cft, hf_new = cell_step(oh_tcr_ref[i], tbl_tf, wtf, hft, cft, mf)
        hbt, cbt, hb_new = cell_step(oh_tcr_ref[tb], tbl_tb, wtb, hbt, cbt, mb)
        tcr_out_ref[:, i, :HIDDEN] = jnp.where(mf, hf_new, 0.0)
        tcr_out_ref[:, tb, HIDDEN:] = jnp.where(mb, hb_new, 0.0)
        return hft, cft, hbt, cbt

    def body_a(i, carry):
        hfp, cfp, hbp, cbp, hft, cft, hbt, cbt = carry
        tb = PEP_LENGTH - 1 - i
        hfp, cfp, _ = cell_step(oh_pep_ref[i], tbl_pf, wpf, hfp, cfp,
                                i < lens_p)
        hbp, cbp, _ = cell_step(oh_pep_ref[tb], tbl_pb, wpb, hbp, cbp,
                                tb < lens_p)
        hft, cft, hbt, cbt = tcr_step(i, hft, cft, hbt, cbt)
        return hfp, cfp, hbp, cbp, hft, cft, hbt, cbt

    h0p, c0p = h0p_ref[...], c0p_ref[...]
    h0t, c0t = h0t_ref[...], c0t_ref[...]
    carry = (h0p, c0p, h0p, c0p, h0t, c0t, h0t, c0t)
    for i in range(PEP_LENGTH):
        carry = body_a(i, carry)
    hfp, _, hbp, _, hft, cft, hbt, cbt = carry
    carry4 = (hft, cft, hbt, cbt)
    for i in range(PEP_LENGTH, MAX_TCR_LEN):
        carry4 = tcr_step(i, *carry4)
    hft, _, hbt, _ = carry4

    pep_emb_ref[:, :HIDDEN] = hfp
    pep_emb_ref[:, HIDDEN:] = hbp
    tcr_hn_ref[0] = hft
    tcr_hn_ref[1] = hbt


@jax.jit
def kernel(obs, emb_table, onehot_dict, pep_Wih_f, pep_Whh_f, pep_b_f,
           pep_Wih_b, pep_Whh_b, pep_b_b, tcr_Wih_f, tcr_Whh_f, tcr_b_f,
           tcr_Wih_b, tcr_Whh_b, tcr_b_b, h0_pep, c0_pep, h0_tcr, c0_tcr):
    B = obs.shape[0]
    Bb = B // NB

    args = (obs.astype(jnp.int32), emb_table, onehot_dict,
            pep_Wih_f, pep_Wih_b, tcr_Wih_f, tcr_Wih_b,
            pep_b_f.reshape(1, G4), pep_b_b.reshape(1, G4),
            tcr_b_f.reshape(1, G4), tcr_b_b.reshape(1, G4),
            pep_Whh_f, pep_Whh_b, tcr_Whh_f, tcr_Whh_b,
            h0_pep, c0_pep, h0_tcr, c0_tcr)

    full = lambda b: (0, 0)
    bat2 = lambda b: (b, 0)
    in_specs = [
        pl.BlockSpec((Bb, TOT_LEN), bat2),
        pl.BlockSpec((VOCAB, 32), full),
        pl.BlockSpec((VOCAB, N_AA), full),
        pl.BlockSpec((G4, ENC_DIM), full),
        pl.BlockSpec((G4, ENC_DIM), full),
        pl.BlockSpec((G4, ENC_DIM), full),
        pl.BlockSpec((G4, ENC_DIM), full),
        pl.BlockSpec((1, G4), full),
        pl.BlockSpec((1, G4), full),
        pl.BlockSpec((1, G4), full),
        pl.BlockSpec((1, G4), full),
        pl.BlockSpec((G4, HIDDEN), full),
        pl.BlockSpec((G4, HIDDEN), full),
        pl.BlockSpec((G4, HIDDEN), full),
        pl.BlockSpec((G4, HIDDEN), full),
        pl.BlockSpec((Bb, HIDDEN), bat2),
        pl.BlockSpec((Bb, HIDDEN), bat2),
        pl.BlockSpec((Bb, HIDDEN), bat2),
        pl.BlockSpec((Bb, HIDDEN), bat2),
    ]
    out_specs = [
        pl.BlockSpec((Bb, MAX_TCR_LEN, 2 * HIDDEN), lambda b: (b, 0, 0)),
        pl.BlockSpec((2, Bb, HIDDEN), lambda b: (0, b, 0)),
        pl.BlockSpec((Bb, 2 * HIDDEN), bat2),
    ]
    out_shapes = [
        jax.ShapeDtypeStruct((B, MAX_TCR_LEN, 2 * HIDDEN), jnp.float32),
        jax.ShapeDtypeStruct((2, B, HIDDEN), jnp.float32),
        jax.ShapeDtypeStruct((B, 2 * HIDDEN), jnp.float32),
    ]
    tcr_out, tcr_hn, pep_emb = pl.pallas_call(
        _seq_kernel,
        grid=(NB,),
        in_specs=in_specs,
        out_specs=out_specs,
        out_shape=out_shapes,
        scratch_shapes=[
            pltpu.VMEM((PEP_LENGTH, Bb, VOC_PAD), jnp.float32),
            pltpu.VMEM((MAX_TCR_LEN, Bb, VOC_PAD), jnp.float32),
        ],
        compiler_params=pltpu.CompilerParams(
            dimension_semantics=("arbitrary",)),
    )(*args)
    return tcr_out, tcr_hn, pep_emb


# speedup vs baseline: 1.2354x; 1.0625x over previous
import jax
import jax.numpy as jnp
from jax.experimental import pallas as pl
from jax.experimental.pallas import tpu as pltpu

HIDDEN = 128
N_AA = 20
PEP_LENGTH = 15
MAX_TCR_LEN = 27
TOT_LEN = MAX_TCR_LEN + PEP_LENGTH
VOCAB = N_AA + 1
ENC_DIM = 32 + N_AA
VOC_PAD = 32
G4 = 4 * HIDDEN
NB = 4


def _sig(x):
    return 0.5 + 0.5 * jnp.tanh(0.5 * x)


def _cell(gates, c):
    i = _sig(gates[:, :HIDDEN])
    f = _sig(gates[:, HIDDEN:2 * HIDDEN])
    g = jnp.tanh(gates[:, 2 * HIDDEN:3 * HIDDEN])
    o = _sig(gates[:, 3 * HIDDEN:])
    c_new = f * c + i * g
    h_new = o * jnp.tanh(c_new)
    return h_new, c_new


def _dot(a, b):
    return jnp.dot(a, b, preferred_element_type=jnp.float32)


def _seq_kernel(obs_ref, emb_ref, onehot_ref,
                wih_pf_ref, wih_pb_ref, wih_tf_ref, wih_tb_ref,
                b_pf_ref, b_pb_ref, b_tf_ref, b_tb_ref,
                whh_pf_ref, whh_pb_ref, whh_tf_ref, whh_tb_ref,
                h0p_ref, c0p_ref, h0t_ref, c0t_ref,
                tcr_out_ref, tcr_hn_ref, pep_emb_ref,
                oh_pep_ref, oh_tcr_ref):
    enc = jnp.concatenate([emb_ref[...], onehot_ref[...]], axis=1)
    enc = jnp.pad(enc, ((0, VOC_PAD - VOCAB), (0, 0)))

    def tbl(wih_ref, b_ref):
        return _dot(enc, wih_ref[...].T) + b_ref[...]

    tbl_pf = tbl(wih_pf_ref, b_pf_ref)
    tbl_pb = tbl(wih_pb_ref, b_pb_ref)
    tbl_tf = tbl(wih_tf_ref, b_tf_ref)
    tbl_tb = tbl(wih_tb_ref, b_tb_ref)

    tcr_tok = obs_ref[:, :MAX_TCR_LEN]
    pep_tok = obs_ref[:, MAX_TCR_LEN:]

    Bb = obs_ref.shape[0]
    pep3 = pep_tok.T.reshape(PEP_LENGTH, Bb, 1)
    iota_p = jax.lax.broadcasted_iota(jnp.int32, (PEP_LENGTH, Bb, VOC_PAD), 2)
    oh_pep_ref[...] = (pep3 == iota_p).astype(jnp.float32)
    tcr3 = tcr_tok.T.reshape(MAX_TCR_LEN, Bb, 1)
    iota_t = jax.lax.broadcasted_iota(jnp.int32, (MAX_TCR_LEN, Bb, VOC_PAD), 2)
    oh_tcr_ref[...] = (tcr3 == iota_t).astype(jnp.float32)

    lens_p = jnp.sum((pep_tok != 0).astype(jnp.int32), axis=1,
                     keepdims=True)
    lens_t = jnp.sum((tcr_tok != 0).astype(jnp.int32), axis=1, keepdims=True)

    def cell_step(oh, tbl_d, w, h, c, m):
        g = _dot(oh, tbl_d) + _dot(h, w)
        h_new, c_new = _cell(g, c)
        return jnp.where(m, h_new, h), jnp.where(m, c_new, c), h_new

    wpf, wpb = whh_pf_ref[...].T, whh_pb_ref[...].T
    wtf, wtb = whh_tf_ref[...].T, whh_tb_ref[...].T

    def tcr_step(i, hft, cft, hbt, cbt):
        tb = MAX_TCR_LEN - 1 - i
        mf = i < lens_t
        mb = tb < lens_t
        hft, cft, hf_new = cell_step(oh_tcr_ref[i], tbl_tf, wtf, hft, cft, mf)
        hbt, cbt, hb_new = cell_step(oh_tcr_ref[tb], tbl_tb, wtb, hbt, cbt, mb)
        tcr_out_ref[:, i, :HIDDEN] = jnp.where(mf, hf_new, 0.0)
        tcr_out_ref[:, tb, HIDDEN:] = jnp.where(mb, hb_new, 0.0)
        return hft, cft, hbt, cbt

    def body_a(i, carry):
        hfp, cfp, hbp, cbp, hft, cft, hbt, cbt = carry
        tb = PEP_LENGTH - 1 - i
        hfp, cfp, _ = cell_step(oh_pep_ref[i], tbl_pf, wpf, hfp, cfp,
                                i < lens_p)
        hbp, cbp, _ = cell_step(oh_pep_ref[tb], tbl_pb, wpb, hbp, cbp,
                                tb < lens_p)
        hft, cft, hbt, cbt = tcr_step(i, hft, cft, hbt, cbt)
        return hfp, cfp, hbp, cbp, hft, cft, hbt, cbt

    h0p, c0p = h0p_ref[...], c0p_ref[...]
    h0t, c0t = h0t_ref[...], c0t_ref[...]
    carry = (h0p, c0p, h0p, c0p, h0t, c0t, h0t, c0t)
    for i in range(PEP_LENGTH):
        carry = body_a(i, carry)
    hfp, _, hbp, _, hft, cft, hbt, cbt = carry
    carry4 = (hft, cft, hbt, cbt)
    for i in range(PEP_LENGTH, MAX_TCR_LEN):
        carry4 = tcr_step(i, *carry4)
    hft, _, hbt, _ = carry4

    pep_emb_ref[:, :HIDDEN] = hfp
    pep_emb_ref[:, HIDDEN:] = hbp
    tcr_hn_ref[0] = hft
    tcr_hn_ref[1] = hbt


@jax.jit
def kernel(obs, emb_table, onehot_dict, pep_Wih_f, pep_Whh_f, pep_b_f,
           pep_Wih_b, pep_Whh_b, pep_b_b, tcr_Wih_f, tcr_Whh_f, tcr_b_f,
           tcr_Wih_b, tcr_Whh_b, tcr_b_b, h0_pep, c0_pep, h0_tcr, c0_tcr):
    B = obs.shape[0]
    Bb = B // NB

    args = (obs.astype(jnp.int32), emb_table, onehot_dict,
            pep_Wih_f, pep_Wih_b, tcr_Wih_f, tcr_Wih_b,
            pep_b_f.reshape(1, G4), pep_b_b.reshape(1, G4),
            tcr_b_f.reshape(1, G4), tcr_b_b.reshape(1, G4),
            pep_Whh_f, pep_Whh_b, tcr_Whh_f, tcr_Whh_b,
            h0_pep, c0_pep, h0_tcr, c0_tcr)

    full = lambda b: (0, 0)
    bat2 = lambda b: (b, 0)
    in_specs = [
        pl.BlockSpec((Bb, TOT_LEN), bat2),
        pl.BlockSpec((VOCAB, 32), full),
        pl.BlockSpec((VOCAB, N_AA), full),
        pl.BlockSpec((G4, ENC_DIM), full),
        pl.BlockSpec((G4, ENC_DIM), full),
        pl.BlockSpec((G4, ENC_DIM), full),
        pl.BlockSpec((G4, ENC_DIM), full),
        pl.BlockSpec((1, G4), full),
        pl.BlockSpec((1, G4), full),
        pl.BlockSpec((1, G4), full),
        pl.BlockSpec((1, G4), full),
        pl.BlockSpec((G4, HIDDEN), full),
        pl.BlockSpec((G4, HIDDEN), full),
        pl.BlockSpec((G4, HIDDEN), full),
        pl.BlockSpec((G4, HIDDEN), full),
        pl.BlockSpec((Bb, HIDDEN), bat2),
        pl.BlockSpec((Bb, HIDDEN), bat2),
        pl.BlockSpec((Bb, HIDDEN), bat2),
        pl.BlockSpec((Bb, HIDDEN), bat2),
    ]
    out_specs = [
        pl.BlockSpec((Bb, MAX_TCR_LEN, 2 * HIDDEN), lambda b: (b, 0, 0)),
        pl.BlockSpec((2, Bb, HIDDEN), lambda b: (0, b, 0)),
        pl.BlockSpec((Bb, 2 * HIDDEN), bat2),
    ]
    out_shapes = [
        jax.ShapeDtypeStruct((B, MAX_TCR_LEN, 2 * HIDDEN), jnp.float32),
        jax.ShapeDtypeStruct((2, B, HIDDEN), jnp.float32),
        jax.ShapeDtypeStruct((B, 2 * HIDDEN), jnp.float32),
    ]
    tcr_out, tcr_hn, pep_emb = pl.pallas_call(
        _seq_kernel,
        grid=(NB,),
        in_specs=in_specs,
        out_specs=out_specs,
        out_shape=out_shapes,
        scratch_shapes=[
            pltpu.VMEM((PEP_LENGTH, Bb, VOC_PAD), jnp.float32),
            pltpu.VMEM((MAX_TCR_LEN, Bb, VOC_PAD), jnp.float32),
        ],
        compiler_params=pltpu.CompilerParams(
            dimension_semantics=("arbitrary",)),
    )(*args)
    return tcr_out, tcr_hn, pep_emb


# speedup vs baseline: 1.2652x; 1.0241x over previous
import jax
import jax.numpy as jnp
from jax.experimental import pallas as pl
from jax.experimental.pallas import tpu as pltpu

HIDDEN = 128
N_AA = 20
PEP_LENGTH = 15
MAX_TCR_LEN = 27
TOT_LEN = MAX_TCR_LEN + PEP_LENGTH
VOCAB = N_AA + 1
ENC_DIM = 32 + N_AA
VOC_PAD = 32
G4 = 4 * HIDDEN
NB = 8


def _sig(x):
    return 0.5 + 0.5 * jnp.tanh(0.5 * x)


def _cell(gates, c):
    i = _sig(gates[:, :HIDDEN])
    f = _sig(gates[:, HIDDEN:2 * HIDDEN])
    g = jnp.tanh(gates[:, 2 * HIDDEN:3 * HIDDEN])
    o = _sig(gates[:, 3 * HIDDEN:])
    c_new = f * c + i * g
    h_new = o * jnp.tanh(c_new)
    return h_new, c_new


def _dot(a, b):
    return jnp.dot(a, b, preferred_element_type=jnp.float32)


def _seq_kernel(obs_ref, emb_ref, onehot_ref,
                wih_pf_ref, wih_pb_ref, wih_tf_ref, wih_tb_ref,
                b_pf_ref, b_pb_ref, b_tf_ref, b_tb_ref,
                whh_pf_ref, whh_pb_ref, whh_tf_ref, whh_tb_ref,
                h0p_ref, c0p_ref, h0t_ref, c0t_ref,
                tcr_out_ref, tcr_hn_ref, pep_emb_ref,
                oh_pep_ref, oh_tcr_ref):
    enc = jnp.concatenate([emb_ref[...], onehot_ref[...]], axis=1)
    enc = jnp.pad(enc, ((0, VOC_PAD - VOCAB), (0, 0)))

    def tbl(wih_ref, b_ref):
        return _dot(enc, wih_ref[...].T) + b_ref[...]

    tbl_pf = tbl(wih_pf_ref, b_pf_ref)
    tbl_pb = tbl(wih_pb_ref, b_pb_ref)
    tbl_tf = tbl(wih_tf_ref, b_tf_ref)
    tbl_tb = tbl(wih_tb_ref, b_tb_ref)

    tcr_tok = obs_ref[:, :MAX_TCR_LEN]
    pep_tok = obs_ref[:, MAX_TCR_LEN:]

    Bb = obs_ref.shape[0]
    pep3 = pep_tok.T.reshape(PEP_LENGTH, Bb, 1)
    iota_p = jax.lax.broadcasted_iota(jnp.int32, (PEP_LENGTH, Bb, VOC_PAD), 2)
    oh_pep_ref[...] = (pep3 == iota_p).astype(jnp.float32)
    tcr3 = tcr_tok.T.reshape(MAX_TCR_LEN, Bb, 1)
    iota_t = jax.lax.broadcasted_iota(jnp.int32, (MAX_TCR_LEN, Bb, VOC_PAD), 2)
    oh_tcr_ref[...] = (tcr3 == iota_t).astype(jnp.float32)

    lens_p = jnp.sum((pep_tok != 0).astype(jnp.int32), axis=1,
                     keepdims=True)
    lens_t = jnp.sum((tcr_tok != 0).astype(jnp.int32), axis=1, keepdims=True)

    def cell_step(oh, tbl_d, w, h, c, m):
        g = _dot(oh, tbl_d) + _dot(h, w)
        h_new, c_new = _cell(g, c)
        return jnp.where(m, h_new, h), jnp.where(m, c_new, c), h_new

    wpf, wpb = whh_pf_ref[...].T, whh_pb_ref[...].T
    wtf, wtb = whh_tf_ref[...].T, whh_tb_ref[...].T

    def tcr_step(i, hft, cft, hbt, cbt):
        tb = MAX_TCR_LEN - 1 - i
        mf = i < lens_t
        mb = tb < lens_t
        hft, cft, hf_new = cell_step(oh_tcr_ref[i], tbl_tf, wtf, hft, cft, mf)
        hbt, cbt, hb_new = cell_step(oh_tcr_ref[tb], tbl_tb, wtb, hbt, cbt, mb)
        tcr_out_ref[:, i, :HIDDEN] = jnp.where(mf, hf_new, 0.0)
        tcr_out_ref[:, tb, HIDDEN:] = jnp.where(mb, hb_new, 0.0)
        return hft, cft, hbt, cbt

    def body_a(i, carry):
        hfp, cfp, hbp, cbp, hft, cft, hbt, cbt = carry
        tb = PEP_LENGTH - 1 - i
        hfp, cfp, _ = cell_step(oh_pep_ref[i], tbl_pf, wpf, hfp, cfp,
                                i < lens_p)
        hbp, cbp, _ = cell_step(oh_pep_ref[tb], tbl_pb, wpb, hbp, cbp,
                                tb < lens_p)
        hft, cft, hbt, cbt = tcr_step(i, hft, cft, hbt, cbt)
        return hfp, cfp, hbp, cbp, hft, cft, hbt, cbt

    h0p, c0p = h0p_ref[...], c0p_ref[...]
    h0t, c0t = h0t_ref[...], c0t_ref[...]
    carry = (h0p, c0p, h0p, c0p, h0t, c0t, h0t, c0t)
    for i in range(PEP_LENGTH):
        carry = body_a(i, carry)
    hfp, _, hbp, _, hft, cft, hbt, cbt = carry
    carry4 = (hft, cft, hbt, cbt)
    for i in range(PEP_LENGTH, MAX_TCR_LEN):
        carry4 = tcr_step(i, *carry4)
    hft, _, hbt, _ = carry4

    pep_emb_ref[:, :HIDDEN] = hfp
    pep_emb_ref[:, HIDDEN:] = hbp
    tcr_hn_ref[0] = hft
    tcr_hn_ref[1] = hbt


@jax.jit
def kernel(obs, emb_table, onehot_dict, pep_Wih_f, pep_Whh_f, pep_b_f,
           pep_Wih_b, pep_Whh_b, pep_b_b, tcr_Wih_f, tcr_Whh_f, tcr_b_f,
           tcr_Wih_b, tcr_Whh_b, tcr_b_b, h0_pep, c0_pep, h0_tcr, c0_tcr):
    B = obs.shape[0]
    Bb = B // NB

    args = (obs.astype(jnp.int32), emb_table, onehot_dict,
            pep_Wih_f, pep_Wih_b, tcr_Wih_f, tcr_Wih_b,
            pep_b_f.reshape(1, G4), pep_b_b.reshape(1, G4),
            tcr_b_f.reshape(1, G4), tcr_b_b.reshape(1, G4),
            pep_Whh_f, pep_Whh_b, tcr_Whh_f, tcr_Whh_b,
            h0_pep, c0_pep, h0_tcr, c0_tcr)

    full = lambda b: (0, 0)
    bat2 = lambda b: (b, 0)
    in_specs = [
        pl.BlockSpec((Bb, TOT_LEN), bat2),
        pl.BlockSpec((VOCAB, 32), full),
        pl.BlockSpec((VOCAB, N_AA), full),
        pl.BlockSpec((G4, ENC_DIM), full),
        pl.BlockSpec((G4, ENC_DIM), full),
        pl.BlockSpec((G4, ENC_DIM), full),
        pl.BlockSpec((G4, ENC_DIM), full),
        pl.BlockSpec((1, G4), full),
        pl.BlockSpec((1, G4), full),
        pl.BlockSpec((1, G4), full),
        pl.BlockSpec((1, G4), full),
        pl.BlockSpec((G4, HIDDEN), full),
        pl.BlockSpec((G4, HIDDEN), full),
        pl.BlockSpec((G4, HIDDEN), full),
        pl.BlockSpec((G4, HIDDEN), full),
        pl.BlockSpec((Bb, HIDDEN), bat2),
        pl.BlockSpec((Bb, HIDDEN), bat2),
        pl.BlockSpec((Bb, HIDDEN), bat2),
        pl.BlockSpec((Bb, HIDDEN), bat2),
    ]
    out_specs = [
        pl.BlockSpec((Bb, MAX_TCR_LEN, 2 * HIDDEN), lambda b: (b, 0, 0)),
        pl.BlockSpec((2, Bb, HIDDEN), lambda b: (0, b, 0)),
        pl.BlockSpec((Bb, 2 * HIDDEN), bat2),
    ]
    out_shapes = [
        jax.ShapeDtypeStruct((B, MAX_TCR_LEN, 2 * HIDDEN), jnp.float32),
        jax.ShapeDtypeStruct((2, B, HIDDEN), jnp.float32),
        jax.ShapeDtypeStruct((B, 2 * HIDDEN), jnp.float32),
    ]
    tcr_out, tcr_hn, pep_emb = pl.pallas_call(
        _seq_kernel,
        grid=(NB,),
        in_specs=in_specs,
        out_specs=out_specs,
        out_shape=out_shapes,
        scratch_shapes=[
            pltpu.VMEM((PEP_LENGTH, Bb, VOC_PAD), jnp.float32),
            pltpu.VMEM((MAX_TCR_LEN, Bb, VOC_PAD), jnp.float32),
        ],
        compiler_params=pltpu.CompilerParams(
            dimension_semantics=("arbitrary",)),
    )(*args)
    return tcr_out, tcr_hn, pep_emb
